# baseline (speedup 1.0000x reference)
import jax
import jax.numpy as jnp
from jax import lax
from jax.experimental import pallas as pl
from jax.experimental.pallas import tpu as pltpu
from jax.experimental.pallas import tpu_sc as plsc

N = 10000
E = 320000
DF = 128
DC = 64
CW = 16
NC = 2
NS = 16
EPT = E // NS
CB = 80
NJF = EPT // CB
NJP = NJF // NC
NBF = 5
NBP = 5
RPS = N // NS


def _fill2(ref, rows, cols, value):
    v = jnp.full((16,), value, jnp.float32)

    @pl.loop(0, rows)
    def _row(i):
        @pl.loop(0, cols // 16)
        def _col(k):
            ref[i, pl.ds(k * 16, 16)] = v


def _seg_common(acc_sh, zb, sid):
    for t in range(RPS // 25):
        pltpu.sync_copy(zb, acc_sh.at[pl.ds(sid * RPS + t * 25, 25)])


def _run_pipeline(x_hbm, gidx_v, dst_v, rows_v, sems, acc_sh, nj, nb,
                  cnt_fn=None):
    ng = nj // nb
    for b in range(nb):
        pltpu.async_copy(x_hbm.at[gidx_v.at[b]], rows_v.at[b], sems[b])

    @pl.loop(0, ng)
    def _group(g):
        for b in range(nb):
            j = g * nb + b
            pltpu.make_async_copy(
                x_hbm.at[gidx_v.at[j]], rows_v.at[b], sems[b]).wait()
            pltpu.sync_copy(rows_v.at[b], acc_sh.at[dst_v.at[j]], add=True)
            if cnt_fn is not None:
                cnt_fn(j)

            @pl.when(g + 1 < ng)
            def _prefetch(b=b, g=g):
                pltpu.async_copy(x_hbm.at[gidx_v.at[(g + 1) * nb + b]],
                                 rows_v.at[b], sems[b])


def _make_segsum_f():
    mesh = plsc.VectorSubcoreMesh(core_axis_name="c", subcore_axis_name="s")
    out_type = [jax.ShapeDtypeStruct((N, DF), jnp.float32),
                jax.ShapeDtypeStruct((N, DF), jnp.float32)]
    scratch = [
        pltpu.VMEM((NJF, CB), jnp.int32),
        pltpu.VMEM((NJF, CB), jnp.int32),
        pltpu.VMEM((NBF, CB, DC), jnp.float32),
        pltpu.VMEM((25, DC), jnp.float32),
        pltpu.VMEM_SHARED((N, DC), jnp.float32),
        pltpu.VMEM((CB, CW), jnp.float32),
        pltpu.VMEM((125, CW), jnp.float32),
        pltpu.VMEM_SHARED((N, CW), jnp.float32),
    ] + [pltpu.SemaphoreType.DMA for _ in range(NBF)]

    def body(x_hbm, src_hbm, dst_hbm, out_hbm, cnt_hbm,
             src_v, dst_v, rows_v, zb, acc_sh, ones_v, zc, cnt_sh,
             *sems):
        cid = lax.axis_index("c")
        sid = lax.axis_index("s")

        pltpu.sync_copy(src_hbm.at[sid], src_v)
        pltpu.sync_copy(dst_hbm.at[sid], dst_v)
        _fill2(zb, 25, DC, 0.0)
        _fill2(ones_v, CB, CW, 1.0)
        _fill2(zc, 125, CW, 0.0)

        @pl.loop(0, NJF)
        def _xf(j):
            @pl.loop(0, CB // 16)
            def _xf16(k, j=j):
                s = src_v[j, pl.ds(k * 16, 16)]
                src_v[j, pl.ds(k * 16, 16)] = s + s + cid

        _seg_common(acc_sh, zb, sid)
        for t in range(RPS // 125):
            pltpu.sync_copy(zc, cnt_sh.at[pl.ds(sid * RPS + t * 125, 125)])
        plsc.subcore_barrier()

        def cnt_fn(j):
            @pl.when((j // NJP) == cid)
            def _():
                pltpu.sync_copy(ones_v, cnt_sh.at[dst_v.at[j]], add=True)

        _run_pipeline(x_hbm, src_v, dst_v, rows_v, sems, acc_sh, NJF, NBF,
                      cnt_fn)

        plsc.subcore_barrier()
        rows_sl = pl.ds(sid * RPS, RPS)
        pltpu.sync_copy(acc_sh.at[rows_sl],
                        out_hbm.at[rows_sl, pl.ds(cid * DC, DC)])
        pltpu.sync_copy(cnt_sh.at[rows_sl],
                        cnt_hbm.at[rows_sl, pl.ds(cid * CW, CW)])

    return pl.kernel(
        body, out_type=out_type, mesh=mesh, scratch_types=scratch,
        compiler_params=pltpu.CompilerParams(use_tc_tiling_on_sc=False))


def _make_segsum_p():
    mesh = plsc.VectorSubcoreMesh(core_axis_name="c", subcore_axis_name="s")
    out_type = [jax.ShapeDtypeStruct((N, DF), jnp.float32)]
    scratch = [
        pltpu.VMEM((NJP, CB), jnp.int32),
        pltpu.VMEM((NJP, CB), jnp.int32),
        pltpu.VMEM((NBP, CB, DC), jnp.float32),
        pltpu.VMEM((25, DC), jnp.float32),
        pltpu.VMEM_SHARED((N, DC), jnp.float32),
    ] + [pltpu.SemaphoreType.DMA for _ in range(NBP)]

    def body(x_hbm, src_hbm, dst_hbm, out_hbm,
             src_v, dst_v, rows_v, zb, acc_sh, *sems):
        cid = lax.axis_index("c")
        sid = lax.axis_index("s")

        pltpu.sync_copy(src_hbm.at[sid, pl.ds(cid * NJP, NJP)], src_v)
        pltpu.sync_copy(dst_hbm.at[sid, pl.ds(cid * NJP, NJP)], dst_v)
        _fill2(zb, 25, DC, 0.0)
        _seg_common(acc_sh, zb, sid)
        plsc.subcore_barrier()

        _run_pipeline(x_hbm, src_v, dst_v, rows_v, sems, acc_sh, NJP, NBP)

        plsc.subcore_barrier()
        rows_sl = pl.ds(sid * RPS, RPS)
        pltpu.sync_copy(acc_sh.at[rows_sl],
                        out_hbm.at[rows_sl, pl.ds(cid * DC, DC)])

    return pl.kernel(
        body, out_type=out_type, mesh=mesh, scratch_types=scratch,
        compiler_params=pltpu.CompilerParams(use_tc_tiling_on_sc=False))


_segsum_f = _make_segsum_f()
_segsum_p = _make_segsum_p()

_BR = 2000
_NBLK = N // _BR


def _dense1(s01, cnt, x, W1l, b1, W1r, W2l, b2, W2r):
    def body(s_ref, c_ref, x_ref, w1l_ref, b1_ref, w1r_ref,
             w2l_ref, b2_ref, w2r_ref, p_ref, r_ref):
        cc = c_ref[...]
        c = jnp.maximum(cc[:, :1] + cc[:, CW:CW + 1], 1.0)
        m = s_ref[...] * (1.0 / c)
        t = (jnp.dot(m, w1l_ref[...], preferred_element_type=jnp.float32)
             + jnp.dot(x_ref[...], w1r_ref[...],
                       preferred_element_type=jnp.float32)
             + b1_ref[...])
        nrm = jnp.sqrt(jnp.sum(t * t, axis=1, keepdims=True))
        h = jnp.maximum(t / jnp.maximum(nrm, 1e-12), 0.0)
        p_ref[...] = jnp.dot(h, w2l_ref[...],
                             preferred_element_type=jnp.float32)
        r_ref[...] = (jnp.dot(h, w2r_ref[...],
                              preferred_element_type=jnp.float32)
                      + b2_ref[...])

    return pl.pallas_call(
        body,
        grid=(_NBLK,),
        in_specs=[
            pl.BlockSpec((_BR, DF), lambda i: (i, 0)),
            pl.BlockSpec((_BR, DF), lambda i: (i, 0)),
            pl.BlockSpec((_BR, DF), lambda i: (i, 0)),
            pl.BlockSpec((DF, DF), lambda i: (0, 0)),
            pl.BlockSpec((1, DF), lambda i: (0, 0)),
            pl.BlockSpec((DF, DF), lambda i: (0, 0)),
            pl.BlockSpec((DF, DC), lambda i: (0, 0)),
            pl.BlockSpec((1, DC), lambda i: (0, 0)),
            pl.BlockSpec((DF, DC), lambda i: (0, 0)),
        ],
        out_specs=[
            pl.BlockSpec((_BR, DC), lambda i: (i, 0)),
            pl.BlockSpec((_BR, DC), lambda i: (i, 0)),
        ],
        out_shape=[
            jax.ShapeDtypeStruct((N, DC), jnp.float32),
            jax.ShapeDtypeStruct((N, DC), jnp.float32),
        ],
    )(s01, cnt, x, W1l, b1, W1r, W2l, b2, W2r)


def _dense2(acc2, cnt, r):
    def body(a_ref, c_ref, r_ref, o_ref):
        cc = c_ref[...]
        c = jnp.maximum(cc[:, :1] + cc[:, CW:CW + 1], 1.0)
        aa = a_ref[...]
        o = (aa[:, :DC] + aa[:, DC:]) / c + r_ref[...]
        nrm = jnp.sqrt(jnp.sum(o * o, axis=1, keepdims=True))
        o = o / jnp.maximum(nrm, 1e-12)
        m = jnp.max(o, axis=1, keepdims=True)
        lse = jnp.log(jnp.sum(jnp.exp(o - m), axis=1, keepdims=True))
        o_ref[...] = o - m - lse

    return pl.pallas_call(
        body,
        grid=(_NBLK,),
        in_specs=[
            pl.BlockSpec((_BR, DF), lambda i: (i, 0)),
            pl.BlockSpec((_BR, DF), lambda i: (i, 0)),
            pl.BlockSpec((_BR, DC), lambda i: (i, 0)),
        ],
        out_specs=pl.BlockSpec((_BR, DC), lambda i: (i, 0)),
        out_shape=jax.ShapeDtypeStruct((N, DC), jnp.float32),
    )(acc2, cnt, r)


def kernel(features, edge_index, W1l, b1, W1r, W2l, b2, W2r):
    src = edge_index[0].reshape(NS, NJF, CB)
    dst = edge_index[1].reshape(NS, NJF, CB)
    x2 = features.reshape(2 * N, DC)
    s01, cnt = _segsum_f(x2, src, dst)
    p, r = _dense1(s01, cnt, features, W1l, b1.reshape(1, DF), W1r,
                   W2l, b2.reshape(1, DC), W2r)
    out = _segsum_p(p, src, dst)
    acc2 = out[0] if isinstance(out, (list, tuple)) else out
    return _dense2(acc2, cnt, r)


# speedup vs baseline: 1.0606x; 1.0606x over previous
import jax
import jax.numpy as jnp
from jax import lax
from jax.experimental import pallas as pl
from jax.experimental.pallas import tpu as pltpu
from jax.experimental.pallas import tpu_sc as plsc

N = 10000
E = 320000
DF = 128
DC = 64
CW = 16
NC = 2
NS = 16
EPT = E // NS
CB = 80
NJF = EPT // CB
NJP = NJF // NC
NB = 5
RPS = N // NS


def _fill2(ref, rows, cols, value):
    v = jnp.full((16,), value, jnp.float32)

    @pl.loop(0, rows)
    def _row(i):
        @pl.loop(0, cols // 16)
        def _col(k):
            ref[i, pl.ds(k * 16, 16)] = v


def _seg_common(acc_sh, zb, sid):
    for t in range(RPS // 25):
        pltpu.sync_copy(zb, acc_sh.at[pl.ds(sid * RPS + t * 25, 25)])


def _run_pipeline(x_hbm, gidx_v, dst_v, rows_v, sems, acc_sh, nj,
                  cnt_fn=None):
    ng = nj // NB
    for b in range(NB):
        pltpu.async_copy(x_hbm.at[gidx_v.at[b]], rows_v.at[b], sems[b])

    @pl.loop(0, ng)
    def _group(g):
        for b in range(NB):
            j = g * NB + b
            pltpu.make_async_copy(
                x_hbm.at[gidx_v.at[j]], rows_v.at[b], sems[b]).wait()
            pltpu.sync_copy(rows_v.at[b], acc_sh.at[dst_v.at[j]], add=True)
            if cnt_fn is not None:
                cnt_fn(j)

            @pl.when(g + 1 < ng)
            def _prefetch(b=b, g=g):
                pltpu.async_copy(x_hbm.at[gidx_v.at[(g + 1) * NB + b]],
                                 rows_v.at[b], sems[b])


def _make_segsum_f():
    mesh = plsc.VectorSubcoreMesh(core_axis_name="c", subcore_axis_name="s")
    out_type = [jax.ShapeDtypeStruct((N, DF), jnp.float32),
                jax.ShapeDtypeStruct((N, DF), jnp.float32)]
    scratch = [
        pltpu.VMEM((NJF, CB), jnp.int32),
        pltpu.VMEM((NJF, CB), jnp.int32),
        pltpu.VMEM((NB, CB, DC), jnp.float32),
        pltpu.VMEM((25, DC), jnp.float32),
        pltpu.VMEM_SHARED((N, DC), jnp.float32),
        pltpu.VMEM((CB, CW), jnp.float32),
        pltpu.VMEM((125, CW), jnp.float32),
        pltpu.VMEM_SHARED((N, CW), jnp.float32),
    ] + [pltpu.SemaphoreType.DMA for _ in range(NB)]

    def body(x_hbm, src_hbm, dst_hbm, out_hbm, cnt_hbm,
             src_v, dst_v, rows_v, zb, acc_sh, ones_v, zc, cnt_sh,
             *sems):
        cid = lax.axis_index("c")
        sid = lax.axis_index("s")

        pltpu.sync_copy(src_hbm.at[sid], src_v)
        pltpu.sync_copy(dst_hbm.at[sid], dst_v)
        _fill2(zb, 25, DC, 0.0)
        _fill2(ones_v, CB, CW, 1.0)
        _fill2(zc, 125, CW, 0.0)

        @pl.loop(0, NJF)
        def _xf(j):
            @pl.loop(0, CB // 16)
            def _xf16(k, j=j):
                s = src_v[j, pl.ds(k * 16, 16)]
                src_v[j, pl.ds(k * 16, 16)] = s + s + cid

        _seg_common(acc_sh, zb, sid)
        for t in range(RPS // 125):
            pltpu.sync_copy(zc, cnt_sh.at[pl.ds(sid * RPS + t * 125, 125)])
        plsc.subcore_barrier()

        def cnt_fn(j):
            @pl.when((j // NJP) == cid)
            def _():
                pltpu.sync_copy(ones_v, cnt_sh.at[dst_v.at[j]], add=True)

        _run_pipeline(x_hbm, src_v, dst_v, rows_v, sems, acc_sh, NJF,
                      cnt_fn)

        plsc.subcore_barrier()
        rows_sl = pl.ds(sid * RPS, RPS)
        pltpu.sync_copy(acc_sh.at[rows_sl],
                        out_hbm.at[rows_sl, pl.ds(cid * DC, DC)])
        pltpu.sync_copy(cnt_sh.at[rows_sl],
                        cnt_hbm.at[rows_sl, pl.ds(cid * CW, CW)])

    return pl.kernel(
        body, out_type=out_type, mesh=mesh, scratch_types=scratch,
        compiler_params=pltpu.CompilerParams(use_tc_tiling_on_sc=False))


def _make_segsum_p():
    mesh = plsc.VectorSubcoreMesh(core_axis_name="c", subcore_axis_name="s")
    out_type = [jax.ShapeDtypeStruct((N, DF), jnp.float32)]
    scratch = [
        pltpu.VMEM((NJP, CB), jnp.int32),
        pltpu.VMEM((NJP, CB), jnp.int32),
        pltpu.VMEM((NB, CB, DC), jnp.float32),
        pltpu.VMEM((25, DC), jnp.float32),
        pltpu.VMEM_SHARED((N, DC), jnp.float32),
    ] + [pltpu.SemaphoreType.DMA for _ in range(NB)]

    def body(x_hbm, src_hbm, dst_hbm, out_hbm,
             src_v, dst_v, rows_v, zb, acc_sh, *sems):
        cid = lax.axis_index("c")
        sid = lax.axis_index("s")

        pltpu.sync_copy(src_hbm.at[sid, pl.ds(cid * NJP, NJP)], src_v)
        pltpu.sync_copy(dst_hbm.at[sid, pl.ds(cid * NJP, NJP)], dst_v)
        _fill2(zb, 25, DC, 0.0)
        _seg_common(acc_sh, zb, sid)
        plsc.subcore_barrier()

        _run_pipeline(x_hbm, src_v, dst_v, rows_v, sems, acc_sh, NJP)

        plsc.subcore_barrier()
        rows_sl = pl.ds(sid * RPS, RPS)
        pltpu.sync_copy(acc_sh.at[rows_sl],
                        out_hbm.at[rows_sl, pl.ds(cid * DC, DC)])

    return pl.kernel(
        body, out_type=out_type, mesh=mesh, scratch_types=scratch,
        compiler_params=pltpu.CompilerParams(use_tc_tiling_on_sc=False))


_segsum_f = _make_segsum_f()
_segsum_p = _make_segsum_p()

_BR = 1000
_NBLK = N // _BR


def _dense0(x, W1r, b1):
    def body(x_ref, w1r_ref, b1_ref, xr_ref):
        xr_ref[...] = jnp.dot(x_ref[...], w1r_ref[...],
                              preferred_element_type=jnp.float32) \
            + b1_ref[...]

    return pl.pallas_call(
        body,
        grid=(_NBLK,),
        in_specs=[
            pl.BlockSpec((_BR, DF), lambda i: (i, 0)),
            pl.BlockSpec((DF, DF), lambda i: (0, 0)),
            pl.BlockSpec((1, DF), lambda i: (0, 0)),
        ],
        out_specs=pl.BlockSpec((_BR, DF), lambda i: (i, 0)),
        out_shape=jax.ShapeDtypeStruct((N, DF), jnp.float32),
    )(x, W1r, b1)


def _dense1(s01, cnt, xr, W1l, W2l, b2, W2r):
    def body(s_ref, c_ref, xr_ref, w1l_ref,
             w2l_ref, b2_ref, w2r_ref, p_ref, r_ref):
        cc = c_ref[...]
        c = jnp.maximum(cc[:, :1] + cc[:, CW:CW + 1], 1.0)
        m = s_ref[...] * (1.0 / c)
        t = (jnp.dot(m, w1l_ref[...], preferred_element_type=jnp.float32)
             + xr_ref[...])
        nrm = jnp.sqrt(jnp.sum(t * t, axis=1, keepdims=True))
        h = jnp.maximum(t / jnp.maximum(nrm, 1e-12), 0.0)
        p_ref[...] = jnp.dot(h, w2l_ref[...],
                             preferred_element_type=jnp.float32)
        r_ref[...] = (jnp.dot(h, w2r_ref[...],
                              preferred_element_type=jnp.float32)
                      + b2_ref[...])

    return pl.pallas_call(
        body,
        grid=(_NBLK,),
        in_specs=[
            pl.BlockSpec((_BR, DF), lambda i: (i, 0)),
            pl.BlockSpec((_BR, DF), lambda i: (i, 0)),
            pl.BlockSpec((_BR, DF), lambda i: (i, 0)),
            pl.BlockSpec((DF, DF), lambda i: (0, 0)),
            pl.BlockSpec((DF, DC), lambda i: (0, 0)),
            pl.BlockSpec((1, DC), lambda i: (0, 0)),
            pl.BlockSpec((DF, DC), lambda i: (0, 0)),
        ],
        out_specs=[
            pl.BlockSpec((_BR, DC), lambda i: (i, 0)),
            pl.BlockSpec((_BR, DC), lambda i: (i, 0)),
        ],
        out_shape=[
            jax.ShapeDtypeStruct((N, DC), jnp.float32),
            jax.ShapeDtypeStruct((N, DC), jnp.float32),
        ],
    )(s01, cnt, xr, W1l, W2l, b2, W2r)


def _dense2(acc2, cnt, r):
    def body(a_ref, c_ref, r_ref, o_ref):
        cc = c_ref[...]
        c = jnp.maximum(cc[:, :1] + cc[:, CW:CW + 1], 1.0)
        aa = a_ref[...]
        o = (aa[:, :DC] + aa[:, DC:]) / c + r_ref[...]
        nrm = jnp.sqrt(jnp.sum(o * o, axis=1, keepdims=True))
        o = o / jnp.maximum(nrm, 1e-12)
        m = jnp.max(o, axis=1, keepdims=True)
        lse = jnp.log(jnp.sum(jnp.exp(o - m), axis=1, keepdims=True))
        o_ref[...] = o - m - lse

    return pl.pallas_call(
        body,
        grid=(_NBLK,),
        in_specs=[
            pl.BlockSpec((_BR, DF), lambda i: (i, 0)),
            pl.BlockSpec((_BR, DF), lambda i: (i, 0)),
            pl.BlockSpec((_BR, DC), lambda i: (i, 0)),
        ],
        out_specs=pl.BlockSpec((_BR, DC), lambda i: (i, 0)),
        out_shape=jax.ShapeDtypeStruct((N, DC), jnp.float32),
    )(acc2, cnt, r)


def kernel(features, edge_index, W1l, b1, W1r, W2l, b2, W2r):
    src = edge_index[0].reshape(NS, NJF, CB)
    dst = edge_index[1].reshape(NS, NJF, CB)
    x2 = features.reshape(2 * N, DC)
    xr = _dense0(features, W1r, b1.reshape(1, DF))
    s01, cnt = _segsum_f(x2, src, dst)
    p, r = _dense1(s01, cnt, xr, W1l,
                   W2l, b2.reshape(1, DC), W2r)
    out = _segsum_p(p, src, dst)
    acc2 = out[0] if isinstance(out, (list, tuple)) else out
    return _dense2(acc2, cnt, r)


# speedup vs baseline: 1.0622x; 1.0015x over previous
import jax
import jax.numpy as jnp
from jax import lax
from jax.experimental import pallas as pl
from jax.experimental.pallas import tpu as pltpu
from jax.experimental.pallas import tpu_sc as plsc

N = 10000
E = 320000
DF = 128
DC = 64
CW = 16
NC = 2
NS = 16
EPT = E // NS
CB = 80
NJF = EPT // CB
NJP = NJF // NC
NB = 5
RPS = N // NS


def _fill2(ref, rows, cols, value):
    v = jnp.full((16,), value, jnp.float32)

    @pl.loop(0, rows)
    def _row(i):
        @pl.loop(0, cols // 16)
        def _col(k):
            ref[i, pl.ds(k * 16, 16)] = v


def _seg_common(acc_sh, zb, sid):
    for t in range(RPS // 25):
        pltpu.sync_copy(zb, acc_sh.at[pl.ds(sid * RPS + t * 25, 25)])


def _run_pipeline(x_hbm, gidx_v, dst_v, rows_v, sems, acc_sh, nj,
                  cnt_fn=None):
    ng = nj // NB
    for b in range(NB):
        pltpu.async_copy(x_hbm.at[gidx_v.at[b]], rows_v.at[b], sems[b])

    @pl.loop(0, ng)
    def _group(g):
        for b in range(NB):
            j = g * NB + b
            pltpu.make_async_copy(
                x_hbm.at[gidx_v.at[j]], rows_v.at[b], sems[b]).wait()
            pltpu.sync_copy(rows_v.at[b], acc_sh.at[dst_v.at[j]], add=True)
            if cnt_fn is not None:
                cnt_fn(j)

            @pl.when(g + 1 < ng)
            def _prefetch(b=b, g=g):
                pltpu.async_copy(x_hbm.at[gidx_v.at[(g + 1) * NB + b]],
                                 rows_v.at[b], sems[b])


def _make_segsum_f():
    mesh = plsc.VectorSubcoreMesh(core_axis_name="c", subcore_axis_name="s")
    out_type = [jax.ShapeDtypeStruct((N, DF), jnp.float32),
                jax.ShapeDtypeStruct((N, DF), jnp.float32)]
    scratch = [
        pltpu.VMEM((NJF, CB), jnp.int32),
        pltpu.VMEM((NJF, CB), jnp.int32),
        pltpu.VMEM((NB, CB, DC), jnp.float32),
        pltpu.VMEM((25, DC), jnp.float32),
        pltpu.VMEM_SHARED((N, DC), jnp.float32),
        pltpu.VMEM((CB, CW), jnp.float32),
        pltpu.VMEM((125, CW), jnp.float32),
        pltpu.VMEM_SHARED((N, CW), jnp.float32),
    ] + [pltpu.SemaphoreType.DMA for _ in range(NB)]

    def body(x_hbm, src_hbm, dst_hbm, out_hbm, cnt_hbm,
             src_v, dst_v, rows_v, zb, acc_sh, ones_v, zc, cnt_sh,
             *sems):
        cid = lax.axis_index("c")
        sid = lax.axis_index("s")

        pltpu.sync_copy(src_hbm.at[sid], src_v)
        pltpu.sync_copy(dst_hbm.at[sid], dst_v)
        _fill2(zb, 25, DC, 0.0)
        _fill2(ones_v, CB, CW, 1.0)
        _fill2(zc, 125, CW, 0.0)

        @pl.loop(0, NJF)
        def _xf(j):
            @pl.loop(0, CB // 16)
            def _xf16(k, j=j):
                s = src_v[j, pl.ds(k * 16, 16)]
                src_v[j, pl.ds(k * 16, 16)] = s + s + cid

        _seg_common(acc_sh, zb, sid)
        for t in range(RPS // 125):
            pltpu.sync_copy(zc, cnt_sh.at[pl.ds(sid * RPS + t * 125, 125)])
        plsc.subcore_barrier()

        def cnt_fn(j):
            @pl.when((j // NJP) == cid)
            def _():
                pltpu.sync_copy(ones_v, cnt_sh.at[dst_v.at[j]], add=True)

        _run_pipeline(x_hbm, src_v, dst_v, rows_v, sems, acc_sh, NJF,
                      cnt_fn)

        plsc.subcore_barrier()
        rows_sl = pl.ds(sid * RPS, RPS)
        pltpu.sync_copy(acc_sh.at[rows_sl],
                        out_hbm.at[rows_sl, pl.ds(cid * DC, DC)])
        pltpu.sync_copy(cnt_sh.at[rows_sl],
                        cnt_hbm.at[rows_sl, pl.ds(cid * CW, CW)])

    return pl.kernel(
        body, out_type=out_type, mesh=mesh, scratch_types=scratch,
        compiler_params=pltpu.CompilerParams(use_tc_tiling_on_sc=False))


def _make_segsum_p():
    mesh = plsc.VectorSubcoreMesh(core_axis_name="c", subcore_axis_name="s")
    out_type = [jax.ShapeDtypeStruct((N, DF), jnp.float32)]
    scratch = [
        pltpu.VMEM((NJP, CB), jnp.int32),
        pltpu.VMEM((NJP, CB), jnp.int32),
        pltpu.VMEM((NB, CB, DC), jnp.float32),
        pltpu.VMEM((25, DC), jnp.float32),
        pltpu.VMEM_SHARED((N, DC), jnp.float32),
    ] + [pltpu.SemaphoreType.DMA for _ in range(NB)]

    def body(x_hbm, src_hbm, dst_hbm, out_hbm,
             src_v, dst_v, rows_v, zb, acc_sh, *sems):
        cid = lax.axis_index("c")
        sid = lax.axis_index("s")

        pltpu.sync_copy(src_hbm.at[sid, pl.ds(cid * NJP, NJP)], src_v)
        pltpu.sync_copy(dst_hbm.at[sid, pl.ds(cid * NJP, NJP)], dst_v)
        _fill2(zb, 25, DC, 0.0)
        _seg_common(acc_sh, zb, sid)
        plsc.subcore_barrier()

        _run_pipeline(x_hbm, src_v, dst_v, rows_v, sems, acc_sh, NJP)

        plsc.subcore_barrier()
        rows_sl = pl.ds(sid * RPS, RPS)
        pltpu.sync_copy(acc_sh.at[rows_sl],
                        out_hbm.at[rows_sl, pl.ds(cid * DC, DC)])

    return pl.kernel(
        body, out_type=out_type, mesh=mesh, scratch_types=scratch,
        compiler_params=pltpu.CompilerParams(use_tc_tiling_on_sc=False))


_segsum_f = _make_segsum_f()
_segsum_p = _make_segsum_p()

_BR = 1000
_NBLK = N // _BR


def _dense1(s01, cnt, x, W1l, b1, W1r, W2l, b2, W2r):
    def body(s_ref, c_ref, x_ref, w1l_ref, b1_ref, w1r_ref,
             w2l_ref, b2_ref, w2r_ref, p_ref, r_ref):
        cc = c_ref[...]
        c = jnp.maximum(cc[:, :1] + cc[:, CW:CW + 1], 1.0)
        m = s_ref[...] * (1.0 / c)
        t = (jnp.dot(m, w1l_ref[...], preferred_element_type=jnp.float32)
             + jnp.dot(x_ref[...], w1r_ref[...],
                       preferred_element_type=jnp.float32)
             + b1_ref[...])
        nrm = jnp.sqrt(jnp.sum(t * t, axis=1, keepdims=True))
        h = jnp.maximum(t / jnp.maximum(nrm, 1e-12), 0.0)
        p_ref[...] = jnp.dot(h, w2l_ref[...],
                             preferred_element_type=jnp.float32)
        r_ref[...] = (jnp.dot(h, w2r_ref[...],
                              preferred_element_type=jnp.float32)
                      + b2_ref[...])

    return pl.pallas_call(
        body,
        grid=(_NBLK,),
        in_specs=[
            pl.BlockSpec((_BR, DF), lambda i: (i, 0)),
            pl.BlockSpec((_BR, DF), lambda i: (i, 0)),
            pl.BlockSpec((_BR, DF), lambda i: (i, 0)),
            pl.BlockSpec((DF, DF), lambda i: (0, 0)),
            pl.BlockSpec((1, DF), lambda i: (0, 0)),
            pl.BlockSpec((DF, DF), lambda i: (0, 0)),
            pl.BlockSpec((DF, DC), lambda i: (0, 0)),
            pl.BlockSpec((1, DC), lambda i: (0, 0)),
            pl.BlockSpec((DF, DC), lambda i: (0, 0)),
        ],
        out_specs=[
            pl.BlockSpec((_BR, DC), lambda i: (i, 0)),
            pl.BlockSpec((_BR, DC), lambda i: (i, 0)),
        ],
        out_shape=[
            jax.ShapeDtypeStruct((N, DC), jnp.float32),
            jax.ShapeDtypeStruct((N, DC), jnp.float32),
        ],
    )(s01, cnt, x, W1l, b1, W1r, W2l, b2, W2r)


def _dense2(acc2, cnt, r):
    def body(a_ref, c_ref, r_ref, o_ref):
        cc = c_ref[...]
        c = jnp.maximum(cc[:, :1] + cc[:, CW:CW + 1], 1.0)
        aa = a_ref[...]
        o = (aa[:, :DC] + aa[:, DC:]) / c + r_ref[...]
        nrm = jnp.sqrt(jnp.sum(o * o, axis=1, keepdims=True))
        o = o / jnp.maximum(nrm, 1e-12)
        m = jnp.max(o, axis=1, keepdims=True)
        lse = jnp.log(jnp.sum(jnp.exp(o - m), axis=1, keepdims=True))
        o_ref[...] = o - m - lse

    return pl.pallas_call(
        body,
        grid=(_NBLK,),
        in_specs=[
            pl.BlockSpec((_BR, DF), lambda i: (i, 0)),
            pl.BlockSpec((_BR, DF), lambda i: (i, 0)),
            pl.BlockSpec((_BR, DC), lambda i: (i, 0)),
        ],
        out_specs=pl.BlockSpec((_BR, DC), lambda i: (i, 0)),
        out_shape=jax.ShapeDtypeStruct((N, DC), jnp.float32),
    )(acc2, cnt, r)


def kernel(features, edge_index, W1l, b1, W1r, W2l, b2, W2r):
    src = edge_index[0].reshape(NS, NJF, CB)
    dst = edge_index[1].reshape(NS, NJF, CB)
    x2 = features.reshape(2 * N, DC)
    s01, cnt = _segsum_f(x2, src, dst)
    p, r = _dense1(s01, cnt, features, W1l, b1.reshape(1, DF), W1r,
                   W2l, b2.reshape(1, DC), W2r)
    out = _segsum_p(p, src, dst)
    acc2 = out[0] if isinstance(out, (list, tuple)) else out
    return _dense2(acc2, cnt, r)
